# baseline (speedup 1.0000x reference)
import functools

import numpy as np
import jax
import jax.numpy as jnp
from jax import lax
from jax.experimental import pallas as pl
from jax.experimental.pallas import tpu as pltpu
from jax.experimental.pallas import tpu_sc as plsc

NUM_EMBED = 8192
EMBED_FEATS = 256
IN_FEATS = 768
BATCH = 8
HW = 24
N_TOK = BATCH * HW * HW
COMMITMENT_COST = 0.25

TN = HW * HW
NT = N_TOK // TN
TE = 1024
NE = NUM_EMBED // TE
LANES = 128
NF = TE // LANES

NW = 32
BPW = N_TOK // NW
CH = BPW // 2
RPT = NUM_EMBED // 16


def _dist_body(x_ref, w_ref, b_ref, e_ref, idx_ref, dmin_ref, esq_s, e2_s):
    @pl.when(pl.program_id(0) == 0)
    def _():
        def prep(j, c):
            e = e_ref[pl.ds(j * TE, TE), :]
            e2_s[pl.ds(j * TE, TE), :] = e * (-2.0)
            esq_s[pl.ds(j * TE, TE)] = jnp.sum(e * e, axis=1)
            return c
        lax.fori_loop(0, NE, prep, 0)

    x = lax.dot_general(x_ref[0], w_ref[...], (((0,), (1,)), ((), ())),
                        preferred_element_type=jnp.float32) + b_ref[...]
    xsq = jnp.sum(x * x, axis=1, keepdims=True)

    def body(j, carry):
        bestv, besti = carry
        m2 = lax.dot_general(x, e2_s[pl.ds(j * TE, TE), :],
                             (((1,), (1,)), ((), ())),
                             preferred_element_type=jnp.float32)
        esq = esq_s[pl.ds(j * TE, TE)]
        for k in range(NF):
            v = (xsq + esq[None, k * LANES:(k + 1) * LANES]) \
                + m2[:, k * LANES:(k + 1) * LANES]
            upd = v < bestv
            bestv = jnp.where(upd, v, bestv)
            besti = jnp.where(upd, j * TE + k * LANES, besti)
        return bestv, besti

    bestv = jnp.full((TN, LANES), jnp.inf, jnp.float32)
    besti = jnp.zeros((TN, LANES), jnp.int32)
    bestv, besti = lax.fori_loop(0, NE, body, (bestv, besti), unroll=2)

    lane_iota = lax.broadcasted_iota(jnp.int32, (TN, LANES), 1)
    flat = besti + lane_iota
    gmin = jnp.min(bestv, axis=1)
    cand = jnp.where(bestv == gmin[:, None], flat, NUM_EMBED)
    idx_ref[0, 0, :] = jnp.min(cand, axis=1).astype(jnp.int32)
    dmin_ref[0, 0, :] = gmin


def _dist_call(x_nat, W, b2, embed, interpret=False):
    return pl.pallas_call(
        _dist_body,
        grid=(NT,),
        in_specs=[
            pl.BlockSpec((1, IN_FEATS, TN), lambda i: (i, 0, 0)),
            pl.BlockSpec((EMBED_FEATS, IN_FEATS), lambda i: (0, 0)),
            pl.BlockSpec((1, EMBED_FEATS), lambda i: (0, 0)),
            pl.BlockSpec((NUM_EMBED, EMBED_FEATS), lambda i: (0, 0)),
        ],
        out_specs=[
            pl.BlockSpec((1, 1, TN), lambda i: (i, 0, 0)),
            pl.BlockSpec((1, 1, TN), lambda i: (i, 0, 0)),
        ],
        out_shape=[
            jax.ShapeDtypeStruct((NT, 1, TN), jnp.int32),
            jax.ShapeDtypeStruct((NT, 1, TN), jnp.float32),
        ],
        scratch_shapes=[
            pltpu.VMEM((NUM_EMBED,), jnp.float32),
            pltpu.VMEM((NUM_EMBED, EMBED_FEATS), jnp.float32),
        ],
        interpret=interpret,
    )(x_nat, W, b2, embed)


def _sc_body(embed_hbm, idx_hbm, zq_hbm, cnt_hbm,
             idx2_v, rows_v, ones_v, zer_v, cnt_sh, sem):
    cid = lax.axis_index("c")
    sid = lax.axis_index("s")
    wid = sid * 2 + cid
    base = wid * BPW
    pltpu.sync_copy(idx_hbm.at[pl.ds(base, CH)], idx2_v.at[0])
    pltpu.sync_copy(idx_hbm.at[pl.ds(base + CH, CH)], idx2_v.at[1])
    c0 = pltpu.async_copy(embed_hbm.at[idx2_v.at[0]],
                          rows_v.at[pl.ds(0, CH)], sem)
    c1 = pltpu.async_copy(embed_hbm.at[idx2_v.at[1]],
                          rows_v.at[pl.ds(CH, CH)], sem)

    def fill_ones(i, c):
        ones_v[i, :] = jnp.ones((16,), jnp.float32)
        return c
    lax.fori_loop(0, CH, fill_ones, 0)

    def fill_zer(i, c):
        zer_v[i, :] = jnp.zeros((16,), jnp.float32)
        return c
    lax.fori_loop(0, RPT, fill_zer, 0)

    pltpu.sync_copy(zer_v, cnt_sh.at[pl.ds(sid * RPT, RPT)])
    plsc.subcore_barrier()
    pltpu.sync_copy(ones_v, cnt_sh.at[idx2_v.at[0]], add=True)
    pltpu.sync_copy(ones_v, cnt_sh.at[idx2_v.at[1]], add=True)
    c0.wait()
    c1.wait()
    pltpu.sync_copy(rows_v, zq_hbm.at[pl.ds(base, BPW)])
    plsc.subcore_barrier()
    pltpu.sync_copy(cnt_sh.at[pl.ds(sid * RPT, RPT)],
                    cnt_hbm.at[cid, pl.ds(sid * RPT, RPT)])


@functools.cache
def _sc_fn():
    return pl.kernel(
        _sc_body,
        out_type=[
            jax.ShapeDtypeStruct((N_TOK, EMBED_FEATS), jnp.float32),
            jax.ShapeDtypeStruct((2, NUM_EMBED, 16), jnp.float32),
        ],
        mesh=plsc.VectorSubcoreMesh(core_axis_name="c", subcore_axis_name="s"),
        compiler_params=pltpu.CompilerParams(use_tc_tiling_on_sc=False),
        scratch_types=[
            pltpu.VMEM((2, CH), jnp.int32),
            pltpu.VMEM((BPW, EMBED_FEATS), jnp.float32),
            pltpu.VMEM((CH, 16), jnp.float32),
            pltpu.VMEM((RPT, 16), jnp.float32),
            pltpu.VMEM_SHARED((NUM_EMBED, 16), jnp.float32),
            pltpu.SemaphoreType.DMA,
        ],
    )


def _fin_body(cnt_ref, dmin_ref, zq_ref, loss_ref, lp_ref, zqt_ref):
    zqt_ref[0] = jnp.transpose(zq_ref[0], (1, 0))

    @pl.when(pl.program_id(0) == 0)
    def _():
        counts = cnt_ref[0] + cnt_ref[1]
        probs = counts / jnp.float32(N_TOK)
        lp = -jnp.sum(probs * jnp.log(probs + 1e-10)) / jnp.float32(16.0)
        loss = COMMITMENT_COST * (jnp.sum(dmin_ref[...]) /
                                  jnp.float32(N_TOK * EMBED_FEATS))
        loss_ref[0, 0] = loss
        lp_ref[0, 0] = lp


def _fin_call(cnt, dmin, zq, interpret=False):
    return pl.pallas_call(
        _fin_body,
        grid=(BATCH,),
        in_specs=[
            pl.BlockSpec((2, NUM_EMBED // 8, 128), lambda i: (0, 0, 0)),
            pl.BlockSpec((N_TOK,), lambda i: (0,)),
            pl.BlockSpec((1, TN, EMBED_FEATS), lambda i: (i, 0, 0)),
        ],
        out_specs=[
            pl.BlockSpec((1, 1), lambda i: (0, 0), memory_space=pltpu.SMEM),
            pl.BlockSpec((1, 1), lambda i: (0, 0), memory_space=pltpu.SMEM),
            pl.BlockSpec((1, EMBED_FEATS, TN), lambda i: (i, 0, 0)),
        ],
        out_shape=[
            jax.ShapeDtypeStruct((1, 1), jnp.float32),
            jax.ShapeDtypeStruct((1, 1), jnp.float32),
            jax.ShapeDtypeStruct((BATCH, EMBED_FEATS, TN), jnp.float32),
        ],
        interpret=interpret,
    )(cnt, dmin, zq)


def kernel(inputs, W, b, embed):
    x_nat = inputs.reshape(BATCH, IN_FEATS, TN)
    q_idx3, dmin3 = _dist_call(x_nat, W, b.reshape(1, EMBED_FEATS), embed)
    q_idx = q_idx3.reshape(N_TOK)
    z_q, cnt = _sc_fn()(embed, q_idx)
    loss, lp, z_q_t = _fin_call(cnt.reshape(2, NUM_EMBED // 8, 128),
                                dmin3.reshape(N_TOK),
                                z_q.reshape(BATCH, TN, EMBED_FEATS))
    z_q_out = z_q_t.reshape(BATCH, EMBED_FEATS, HW, HW)
    num_spatial_positions = N_TOK / BATCH
    kldiv_r = np.log(NUM_EMBED) * num_spatial_positions * jnp.ones(
        (BATCH, 1), dtype=jnp.float32)
    return z_q_out, loss.reshape(()), kldiv_r, lp.reshape(())


# speedup vs baseline: 1.1037x; 1.1037x over previous
import functools

import numpy as np
import jax
import jax.numpy as jnp
from jax import lax
from jax.experimental import pallas as pl
from jax.experimental.pallas import tpu as pltpu
from jax.experimental.pallas import tpu_sc as plsc

NUM_EMBED = 8192
EMBED_FEATS = 256
IN_FEATS = 768
BATCH = 8
HW = 24
N_TOK = BATCH * HW * HW
COMMITMENT_COST = 0.25

TN = HW * HW
NT = N_TOK // TN
TE = 1024
NE = NUM_EMBED // TE
LANES = 128
NF = TE // LANES

NW = 32
BPW = N_TOK // NW
CH = BPW // 2
RPT = NUM_EMBED // 16


def _dist_body(x_ref, w_ref, b_ref, e_ref, idx_ref, dmin_ref, esq_s, e2_s):
    @pl.when(pl.program_id(0) == 0)
    def _():
        def prep(j, c):
            e = e_ref[pl.ds(j * TE, TE), :]
            e2_s[pl.ds(j * TE, TE), :] = e * (-2.0)
            esq_s[pl.ds(j * TE, TE)] = jnp.sum(e * e, axis=1)
            return c
        lax.fori_loop(0, NE, prep, 0)

    x = lax.dot_general(x_ref[0], w_ref[...], (((0,), (1,)), ((), ())),
                        preferred_element_type=jnp.float32) + b_ref[...]
    xsq = jnp.sum(x * x, axis=1, keepdims=True)

    def body(j, carry):
        bestv, besti = carry
        m2 = lax.dot_general(x, e2_s[pl.ds(j * TE, TE), :],
                             (((1,), (1,)), ((), ())),
                             preferred_element_type=jnp.float32)
        esq = esq_s[pl.ds(j * TE, TE)]
        for k in range(NF):
            v = (xsq + esq[None, k * LANES:(k + 1) * LANES]) \
                + m2[:, k * LANES:(k + 1) * LANES]
            upd = v < bestv
            bestv = jnp.where(upd, v, bestv)
            besti = jnp.where(upd, j * TE + k * LANES, besti)
        return bestv, besti

    bestv = jnp.full((TN, LANES), jnp.inf, jnp.float32)
    besti = jnp.zeros((TN, LANES), jnp.int32)
    bestv, besti = lax.fori_loop(0, NE, body, (bestv, besti), unroll=2)

    lane_iota = lax.broadcasted_iota(jnp.int32, (TN, LANES), 1)
    flat = besti + lane_iota
    gmin = jnp.min(bestv, axis=1)
    cand = jnp.where(bestv == gmin[:, None], flat, NUM_EMBED)
    idx_ref[0, 0, :] = jnp.min(cand, axis=1).astype(jnp.int32)
    dmin_ref[0, 0, :] = gmin


def _dist_call(x_nat, W, b2, embed, interpret=False):
    return pl.pallas_call(
        _dist_body,
        grid=(NT,),
        in_specs=[
            pl.BlockSpec((1, IN_FEATS, TN), lambda i: (i, 0, 0)),
            pl.BlockSpec((EMBED_FEATS, IN_FEATS), lambda i: (0, 0)),
            pl.BlockSpec((1, EMBED_FEATS), lambda i: (0, 0)),
            pl.BlockSpec((NUM_EMBED, EMBED_FEATS), lambda i: (0, 0)),
        ],
        out_specs=[
            pl.BlockSpec((1, 1, TN), lambda i: (i, 0, 0)),
            pl.BlockSpec((1, 1, TN), lambda i: (i, 0, 0)),
        ],
        out_shape=[
            jax.ShapeDtypeStruct((NT, 1, TN), jnp.int32),
            jax.ShapeDtypeStruct((NT, 1, TN), jnp.float32),
        ],
        scratch_shapes=[
            pltpu.VMEM((NUM_EMBED,), jnp.float32),
            pltpu.VMEM((NUM_EMBED, EMBED_FEATS), jnp.float32),
        ],
        interpret=interpret,
    )(x_nat, W, b2, embed)


def _sc_body(embed_hbm, idx_hbm, zq_hbm, cnt_hbm,
             idx2_v, rows_v, ones_v, zer_v, cnt_sh, sem):
    cid = lax.axis_index("c")
    sid = lax.axis_index("s")
    wid = sid * 2 + cid
    base = wid * BPW
    pltpu.sync_copy(idx_hbm.at[pl.ds(base, CH)], idx2_v.at[0])
    pltpu.sync_copy(idx_hbm.at[pl.ds(base + CH, CH)], idx2_v.at[1])
    c0 = pltpu.async_copy(embed_hbm.at[idx2_v.at[0]],
                          rows_v.at[pl.ds(0, CH)], sem)
    c1 = pltpu.async_copy(embed_hbm.at[idx2_v.at[1]],
                          rows_v.at[pl.ds(CH, CH)], sem)

    def fill_ones(i, c):
        ones_v[i, :] = jnp.ones((16,), jnp.float32)
        return c
    lax.fori_loop(0, CH, fill_ones, 0)

    def fill_zer(i, c):
        zer_v[i, :] = jnp.zeros((16,), jnp.float32)
        return c
    lax.fori_loop(0, RPT, fill_zer, 0)

    pltpu.sync_copy(zer_v, cnt_sh.at[pl.ds(sid * RPT, RPT)])
    plsc.subcore_barrier()
    pltpu.sync_copy(ones_v, cnt_sh.at[idx2_v.at[0]], add=True)
    pltpu.sync_copy(ones_v, cnt_sh.at[idx2_v.at[1]], add=True)
    c0.wait()
    c1.wait()
    pltpu.sync_copy(rows_v, zq_hbm.at[pl.ds(base, BPW)])
    plsc.subcore_barrier()
    pltpu.sync_copy(cnt_sh.at[pl.ds(sid * RPT, RPT)],
                    cnt_hbm.at[cid, pl.ds(sid * RPT, RPT)])


@functools.cache
def _sc_fn():
    return pl.kernel(
        _sc_body,
        out_type=[
            jax.ShapeDtypeStruct((N_TOK, EMBED_FEATS), jnp.float32),
            jax.ShapeDtypeStruct((2, NUM_EMBED, 16), jnp.float32),
        ],
        mesh=plsc.VectorSubcoreMesh(core_axis_name="c", subcore_axis_name="s"),
        compiler_params=pltpu.CompilerParams(use_tc_tiling_on_sc=False),
        scratch_types=[
            pltpu.VMEM((2, CH), jnp.int32),
            pltpu.VMEM((BPW, EMBED_FEATS), jnp.float32),
            pltpu.VMEM((CH, 16), jnp.float32),
            pltpu.VMEM((RPT, 16), jnp.float32),
            pltpu.VMEM_SHARED((NUM_EMBED, 16), jnp.float32),
            pltpu.SemaphoreType.DMA,
        ],
    )


def _fin_body(cnt_ref, dmin_ref, loss_ref, lp_ref):
    counts = cnt_ref[0] + cnt_ref[1]
    probs = counts / jnp.float32(N_TOK)
    lp = -jnp.sum(probs * jnp.log(probs + 1e-10)) / jnp.float32(16.0)
    loss = COMMITMENT_COST * (jnp.sum(dmin_ref[...]) /
                              jnp.float32(N_TOK * EMBED_FEATS))
    loss_ref[0, 0] = loss
    lp_ref[0, 0] = lp


def _fin_call(cnt, dmin, interpret=False):
    return pl.pallas_call(
        _fin_body,
        in_specs=[
            pl.BlockSpec(memory_space=pltpu.VMEM),
            pl.BlockSpec(memory_space=pltpu.VMEM),
        ],
        out_specs=[
            pl.BlockSpec(memory_space=pltpu.SMEM),
            pl.BlockSpec(memory_space=pltpu.SMEM),
        ],
        out_shape=[
            jax.ShapeDtypeStruct((1, 1), jnp.float32),
            jax.ShapeDtypeStruct((1, 1), jnp.float32),
        ],
        interpret=interpret,
    )(cnt, dmin)


def kernel(inputs, W, b, embed):
    x_nat = inputs.reshape(BATCH, IN_FEATS, TN)
    q_idx3, dmin3 = _dist_call(x_nat, W, b.reshape(1, EMBED_FEATS), embed)
    q_idx = q_idx3.reshape(N_TOK)
    z_q, cnt = _sc_fn()(embed, q_idx)
    loss, lp = _fin_call(cnt.reshape(2, NUM_EMBED // 8, 128),
                         dmin3.reshape(N_TOK))
    z_q_out = jnp.transpose(z_q.reshape(BATCH, HW, HW, EMBED_FEATS),
                            (0, 3, 1, 2))
    num_spatial_positions = N_TOK / BATCH
    kldiv_r = np.log(NUM_EMBED) * num_spatial_positions * jnp.ones(
        (BATCH, 1), dtype=jnp.float32)
    return z_q_out, loss.reshape(()), kldiv_r, lp.reshape(())


# speedup vs baseline: 1.1669x; 1.0573x over previous
import functools

import numpy as np
import jax
import jax.numpy as jnp
from jax import lax
from jax.experimental import pallas as pl
from jax.experimental.pallas import tpu as pltpu
from jax.experimental.pallas import tpu_sc as plsc

NUM_EMBED = 8192
EMBED_FEATS = 256
IN_FEATS = 768
BATCH = 8
HW = 24
N_TOK = BATCH * HW * HW
COMMITMENT_COST = 0.25

TN = HW * HW
NT = N_TOK // TN
TE = 1024
NE = NUM_EMBED // TE
LANES = 128
NF = TE // LANES

NW = 32
BPW = N_TOK // NW
CH = BPW // 2
RPT = NUM_EMBED // 16


def _dist_body(x_ref, w_ref, b_ref, e_ref, idx_ref, dmin_ref, esq_s, e2_s):
    @pl.when(pl.program_id(0) == 0)
    def _():
        def prep(j, c):
            e = e_ref[pl.ds(j * TE, TE), :]
            e2_s[pl.ds(j * TE, TE), :] = e * (-2.0)
            esq_s[pl.ds(j * TE, TE)] = jnp.sum(e * e, axis=1)
            return c
        lax.fori_loop(0, NE, prep, 0)

    x = lax.dot_general(x_ref[0], w_ref[...], (((0,), (1,)), ((), ())),
                        preferred_element_type=jnp.float32) + b_ref[...]
    xsq = jnp.sum(x * x, axis=1, keepdims=True)

    def body(j, carry):
        bestv, besti = carry
        m2 = lax.dot_general(x, e2_s[pl.ds(j * TE, TE), :],
                             (((1,), (1,)), ((), ())),
                             preferred_element_type=jnp.float32)
        esq = esq_s[pl.ds(j * TE, TE)]
        for k in range(NF):
            v = (xsq + esq[None, k * LANES:(k + 1) * LANES]) \
                + m2[:, k * LANES:(k + 1) * LANES]
            upd = v < bestv
            bestv = jnp.where(upd, v, bestv)
            besti = jnp.where(upd, j * TE + k * LANES, besti)
        return bestv, besti

    bestv = jnp.full((TN, LANES), jnp.inf, jnp.float32)
    besti = jnp.zeros((TN, LANES), jnp.int32)
    bestv, besti = lax.fori_loop(0, NE, body, (bestv, besti), unroll=2)

    lane_iota = lax.broadcasted_iota(jnp.int32, (TN, LANES), 1)
    flat = besti + lane_iota
    gmin = jnp.min(bestv, axis=1)
    cand = jnp.where(bestv == gmin[:, None], flat, NUM_EMBED)
    idx_ref[0, 0, :] = jnp.min(cand, axis=1).astype(jnp.int32)
    dmin_ref[0, 0, :] = gmin


def _dist_call(x_nat, W, b2, embed, interpret=False):
    return pl.pallas_call(
        _dist_body,
        grid=(NT,),
        in_specs=[
            pl.BlockSpec((1, IN_FEATS, TN), lambda i: (i, 0, 0)),
            pl.BlockSpec((EMBED_FEATS, IN_FEATS), lambda i: (0, 0)),
            pl.BlockSpec((1, EMBED_FEATS), lambda i: (0, 0)),
            pl.BlockSpec((NUM_EMBED, EMBED_FEATS), lambda i: (0, 0)),
        ],
        out_specs=[
            pl.BlockSpec((1, 1, TN), lambda i: (i, 0, 0)),
            pl.BlockSpec((1, 1, TN), lambda i: (i, 0, 0)),
        ],
        out_shape=[
            jax.ShapeDtypeStruct((NT, 1, TN), jnp.int32),
            jax.ShapeDtypeStruct((NT, 1, TN), jnp.float32),
        ],
        scratch_shapes=[
            pltpu.VMEM((NUM_EMBED,), jnp.float32),
            pltpu.VMEM((NUM_EMBED, EMBED_FEATS), jnp.float32),
        ],
        interpret=interpret,
    )(x_nat, W, b2, embed)


def _sc_gather_body(embed_hbm, idx_hbm, zq_hbm, idx2_v, rows_v, sem):
    cid = lax.axis_index("c")
    sid = lax.axis_index("s")
    wid = sid * 2 + cid
    base = wid * BPW
    pltpu.sync_copy(idx_hbm.at[pl.ds(base, CH)], idx2_v.at[0])
    pltpu.sync_copy(idx_hbm.at[pl.ds(base + CH, CH)], idx2_v.at[1])
    c0 = pltpu.async_copy(embed_hbm.at[idx2_v.at[0]],
                          rows_v.at[pl.ds(0, CH)], sem)
    c1 = pltpu.async_copy(embed_hbm.at[idx2_v.at[1]],
                          rows_v.at[pl.ds(CH, CH)], sem)
    c0.wait()
    c1.wait()
    pltpu.sync_copy(rows_v, zq_hbm.at[pl.ds(base, BPW)])


@functools.cache
def _sc_gather_fn():
    return pl.kernel(
        _sc_gather_body,
        out_type=jax.ShapeDtypeStruct((N_TOK, EMBED_FEATS), jnp.float32),
        mesh=plsc.VectorSubcoreMesh(core_axis_name="c", subcore_axis_name="s"),
        scratch_types=[
            pltpu.VMEM((2, CH), jnp.int32),
            pltpu.VMEM((BPW, EMBED_FEATS), jnp.float32),
            pltpu.SemaphoreType.DMA,
        ],
    )


def _sc_hist_body(idx_hbm, cnt_hbm, idx2_v, ones_v, zer_v, cnt_sh, sem):
    cid = lax.axis_index("c")
    sid = lax.axis_index("s")
    wid = sid * 2 + cid
    base = wid * BPW
    pltpu.sync_copy(idx_hbm.at[pl.ds(base, CH)], idx2_v.at[0])
    pltpu.sync_copy(idx_hbm.at[pl.ds(base + CH, CH)], idx2_v.at[1])

    def fill_ones(i, c):
        ones_v[i, :] = jnp.ones((16,), jnp.float32)
        return c
    lax.fori_loop(0, CH, fill_ones, 0)

    def fill_zer(i, c):
        zer_v[i, :] = jnp.zeros((16,), jnp.float32)
        return c
    lax.fori_loop(0, RPT, fill_zer, 0)

    pltpu.sync_copy(zer_v, cnt_sh.at[pl.ds(sid * RPT, RPT)])
    plsc.subcore_barrier()
    pltpu.sync_copy(ones_v, cnt_sh.at[idx2_v.at[0]], add=True)
    pltpu.sync_copy(ones_v, cnt_sh.at[idx2_v.at[1]], add=True)
    plsc.subcore_barrier()
    pltpu.sync_copy(cnt_sh.at[pl.ds(sid * RPT, RPT)],
                    cnt_hbm.at[cid, pl.ds(sid * RPT, RPT)])


@functools.cache
def _sc_hist_fn():
    return pl.kernel(
        _sc_hist_body,
        out_type=jax.ShapeDtypeStruct((2, NUM_EMBED, 16), jnp.float32),
        mesh=plsc.VectorSubcoreMesh(core_axis_name="c", subcore_axis_name="s"),
        compiler_params=pltpu.CompilerParams(use_tc_tiling_on_sc=False),
        scratch_types=[
            pltpu.VMEM((2, CH), jnp.int32),
            pltpu.VMEM((CH, 16), jnp.float32),
            pltpu.VMEM((RPT, 16), jnp.float32),
            pltpu.VMEM_SHARED((NUM_EMBED, 16), jnp.float32),
            pltpu.SemaphoreType.DMA,
        ],
    )


def _fin_body(cnt_ref, dmin_ref, loss_ref, lp_ref):
    counts = cnt_ref[0] + cnt_ref[1]
    probs = counts / jnp.float32(N_TOK)
    lp = -jnp.sum(probs * jnp.log(probs + 1e-10)) / jnp.float32(16.0)
    loss = COMMITMENT_COST * (jnp.sum(dmin_ref[...]) /
                              jnp.float32(N_TOK * EMBED_FEATS))
    loss_ref[0, 0] = loss
    lp_ref[0, 0] = lp


def _fin_call(cnt, dmin, interpret=False):
    return pl.pallas_call(
        _fin_body,
        in_specs=[
            pl.BlockSpec(memory_space=pltpu.VMEM),
            pl.BlockSpec(memory_space=pltpu.VMEM),
        ],
        out_specs=[
            pl.BlockSpec(memory_space=pltpu.SMEM),
            pl.BlockSpec(memory_space=pltpu.SMEM),
        ],
        out_shape=[
            jax.ShapeDtypeStruct((1, 1), jnp.float32),
            jax.ShapeDtypeStruct((1, 1), jnp.float32),
        ],
        interpret=interpret,
    )(cnt, dmin)


def kernel(inputs, W, b, embed):
    x_nat = inputs.reshape(BATCH, IN_FEATS, TN)
    q_idx3, dmin3 = _dist_call(x_nat, W, b.reshape(1, EMBED_FEATS), embed)
    q_idx = q_idx3.reshape(N_TOK)
    z_q = _sc_gather_fn()(embed, q_idx)
    cnt = _sc_hist_fn()(q_idx)
    loss, lp = _fin_call(cnt.reshape(2, NUM_EMBED // 8, 128),
                         dmin3.reshape(N_TOK))
    z_q_out = jnp.transpose(z_q.reshape(BATCH, HW, HW, EMBED_FEATS),
                            (0, 3, 1, 2))
    num_spatial_positions = N_TOK / BATCH
    kldiv_r = np.log(NUM_EMBED) * num_spatial_positions * jnp.ones(
        (BATCH, 1), dtype=jnp.float32)
    return z_q_out, loss.reshape(()), kldiv_r, lp.reshape(())


# speedup vs baseline: 1.2160x; 1.0421x over previous
import functools

import numpy as np
import jax
import jax.numpy as jnp
from jax import lax
from jax.experimental import pallas as pl
from jax.experimental.pallas import tpu as pltpu
from jax.experimental.pallas import tpu_sc as plsc

NUM_EMBED = 8192
EMBED_FEATS = 256
IN_FEATS = 768
BATCH = 8
HW = 24
N_TOK = BATCH * HW * HW
COMMITMENT_COST = 0.25

TN = HW * HW
NT = N_TOK // TN
TE = 1024
NE = NUM_EMBED // TE
LANES = 128
NF = TE // LANES

NW = 32
BPW = N_TOK // NW
CH = BPW // 2
RPT = NUM_EMBED // 16


def _dist_body(x_ref, w_ref, b_ref, e_ref, idx_ref, dmin_ref, esq_s, e2_s):
    @pl.when(pl.program_id(0) == 0)
    def _():
        def prep(j, c):
            e = e_ref[pl.ds(j * TE, TE), :]
            e2_s[pl.ds(j * TE, TE), :] = e * (-2.0)
            esq_s[pl.ds(j * TE, TE)] = jnp.sum(e * e, axis=1)
            return c
        lax.fori_loop(0, NE, prep, 0)

    x = lax.dot_general(x_ref[0], w_ref[...], (((0,), (1,)), ((), ())),
                        preferred_element_type=jnp.float32) + b_ref[...]
    xsq = jnp.sum(x * x, axis=1, keepdims=True)

    def body(j, carry):
        bestv, besti = carry
        m2 = lax.dot_general(x, e2_s[pl.ds(j * TE, TE), :],
                             (((1,), (1,)), ((), ())),
                             preferred_element_type=jnp.float32)
        esq = esq_s[pl.ds(j * TE, TE)]
        for k in range(NF):
            v = (xsq + esq[None, k * LANES:(k + 1) * LANES]) \
                + m2[:, k * LANES:(k + 1) * LANES]
            upd = v < bestv
            bestv = jnp.where(upd, v, bestv)
            besti = jnp.where(upd, j * TE + k * LANES, besti)
        return bestv, besti

    bestv = jnp.full((TN, LANES), jnp.inf, jnp.float32)
    besti = jnp.zeros((TN, LANES), jnp.int32)
    bestv, besti = lax.fori_loop(0, NE, body, (bestv, besti), unroll=4)

    lane_iota = lax.broadcasted_iota(jnp.int32, (TN, LANES), 1)
    flat = besti + lane_iota
    gmin = jnp.min(bestv, axis=1)
    cand = jnp.where(bestv == gmin[:, None], flat, NUM_EMBED)
    idx_ref[0, 0, :] = jnp.min(cand, axis=1).astype(jnp.int32)
    dmin_ref[0, 0, :] = gmin


def _dist_call(x_nat, W, b2, embed, interpret=False):
    return pl.pallas_call(
        _dist_body,
        grid=(NT,),
        in_specs=[
            pl.BlockSpec((1, IN_FEATS, TN), lambda i: (i, 0, 0)),
            pl.BlockSpec((EMBED_FEATS, IN_FEATS), lambda i: (0, 0)),
            pl.BlockSpec((1, EMBED_FEATS), lambda i: (0, 0)),
            pl.BlockSpec((NUM_EMBED, EMBED_FEATS), lambda i: (0, 0)),
        ],
        out_specs=[
            pl.BlockSpec((1, 1, TN), lambda i: (i, 0, 0)),
            pl.BlockSpec((1, 1, TN), lambda i: (i, 0, 0)),
        ],
        out_shape=[
            jax.ShapeDtypeStruct((NT, 1, TN), jnp.int32),
            jax.ShapeDtypeStruct((NT, 1, TN), jnp.float32),
        ],
        scratch_shapes=[
            pltpu.VMEM((NUM_EMBED,), jnp.float32),
            pltpu.VMEM((NUM_EMBED, EMBED_FEATS), jnp.float32),
        ],
        interpret=interpret,
    )(x_nat, W, b2, embed)


def _sc_gather_body(embed_hbm, idx_hbm, zq_hbm, idx2_v, rows_v, sem):
    cid = lax.axis_index("c")
    sid = lax.axis_index("s")
    wid = sid * 2 + cid
    base = wid * BPW
    pltpu.sync_copy(idx_hbm.at[pl.ds(base, CH)], idx2_v.at[0])
    pltpu.sync_copy(idx_hbm.at[pl.ds(base + CH, CH)], idx2_v.at[1])
    c0 = pltpu.async_copy(embed_hbm.at[idx2_v.at[0]],
                          rows_v.at[pl.ds(0, CH)], sem)
    c1 = pltpu.async_copy(embed_hbm.at[idx2_v.at[1]],
                          rows_v.at[pl.ds(CH, CH)], sem)
    c0.wait()
    c1.wait()
    pltpu.sync_copy(rows_v, zq_hbm.at[pl.ds(base, BPW)])


@functools.cache
def _sc_gather_fn():
    return pl.kernel(
        _sc_gather_body,
        out_type=jax.ShapeDtypeStruct((N_TOK, EMBED_FEATS), jnp.float32),
        mesh=plsc.VectorSubcoreMesh(core_axis_name="c", subcore_axis_name="s"),
        scratch_types=[
            pltpu.VMEM((2, CH), jnp.int32),
            pltpu.VMEM((BPW, EMBED_FEATS), jnp.float32),
            pltpu.SemaphoreType.DMA,
        ],
    )


def _sc_hist_body(idx_hbm, cnt_hbm, idx2_v, ones_v, zer_v, cnt_sh, sem):
    cid = lax.axis_index("c")
    sid = lax.axis_index("s")
    wid = sid * 2 + cid
    base = wid * BPW
    pltpu.sync_copy(idx_hbm.at[pl.ds(base, CH)], idx2_v.at[0])
    pltpu.sync_copy(idx_hbm.at[pl.ds(base + CH, CH)], idx2_v.at[1])

    def fill_ones(i, c):
        ones_v[i, :] = jnp.ones((16,), jnp.float32)
        return c
    lax.fori_loop(0, CH, fill_ones, 0)

    def fill_zer(i, c):
        zer_v[i, :] = jnp.zeros((16,), jnp.float32)
        return c
    lax.fori_loop(0, RPT, fill_zer, 0)

    pltpu.sync_copy(zer_v, cnt_sh.at[pl.ds(sid * RPT, RPT)])
    plsc.subcore_barrier()
    pltpu.sync_copy(ones_v, cnt_sh.at[idx2_v.at[0]], add=True)
    pltpu.sync_copy(ones_v, cnt_sh.at[idx2_v.at[1]], add=True)
    plsc.subcore_barrier()
    pltpu.sync_copy(cnt_sh.at[pl.ds(sid * RPT, RPT)],
                    cnt_hbm.at[cid, pl.ds(sid * RPT, RPT)])


@functools.cache
def _sc_hist_fn():
    return pl.kernel(
        _sc_hist_body,
        out_type=jax.ShapeDtypeStruct((2, NUM_EMBED, 16), jnp.float32),
        mesh=plsc.VectorSubcoreMesh(core_axis_name="c", subcore_axis_name="s"),
        compiler_params=pltpu.CompilerParams(use_tc_tiling_on_sc=False),
        scratch_types=[
            pltpu.VMEM((2, CH), jnp.int32),
            pltpu.VMEM((CH, 16), jnp.float32),
            pltpu.VMEM((RPT, 16), jnp.float32),
            pltpu.VMEM_SHARED((NUM_EMBED, 16), jnp.float32),
            pltpu.SemaphoreType.DMA,
        ],
    )


def _fin_body(cnt_ref, dmin_ref, loss_ref, lp_ref):
    counts = cnt_ref[0] + cnt_ref[1]
    probs = counts / jnp.float32(N_TOK)
    lp = -jnp.sum(probs * jnp.log(probs + 1e-10)) / jnp.float32(16.0)
    loss = COMMITMENT_COST * (jnp.sum(dmin_ref[...]) /
                              jnp.float32(N_TOK * EMBED_FEATS))
    loss_ref[0, 0] = loss
    lp_ref[0, 0] = lp


def _fin_call(cnt, dmin, interpret=False):
    return pl.pallas_call(
        _fin_body,
        in_specs=[
            pl.BlockSpec(memory_space=pltpu.VMEM),
            pl.BlockSpec(memory_space=pltpu.VMEM),
        ],
        out_specs=[
            pl.BlockSpec(memory_space=pltpu.SMEM),
            pl.BlockSpec(memory_space=pltpu.SMEM),
        ],
        out_shape=[
            jax.ShapeDtypeStruct((1, 1), jnp.float32),
            jax.ShapeDtypeStruct((1, 1), jnp.float32),
        ],
        interpret=interpret,
    )(cnt, dmin)


def kernel(inputs, W, b, embed):
    x_nat = inputs.reshape(BATCH, IN_FEATS, TN)
    q_idx3, dmin3 = _dist_call(x_nat, W, b.reshape(1, EMBED_FEATS), embed)
    q_idx = q_idx3.reshape(N_TOK)
    z_q = _sc_gather_fn()(embed, q_idx)
    cnt = _sc_hist_fn()(q_idx)
    loss, lp = _fin_call(cnt.reshape(2, NUM_EMBED // 8, 128),
                         dmin3.reshape(N_TOK))
    z_q_out = jnp.transpose(z_q.reshape(BATCH, HW, HW, EMBED_FEATS),
                            (0, 3, 1, 2))
    num_spatial_positions = N_TOK / BATCH
    kldiv_r = np.log(NUM_EMBED) * num_spatial_positions * jnp.ones(
        (BATCH, 1), dtype=jnp.float32)
    return z_q_out, loss.reshape(()), kldiv_r, lp.reshape(())
